# baseline (speedup 1.0000x reference)
import functools

import jax
import jax.numpy as jnp
from jax import lax
from jax.experimental import pallas as pl
from jax.experimental.pallas import tpu as pltpu
from jax.experimental.pallas import tpu_sc as plsc

BN_EPS = 1e-5
NC = 2
NS = 16


def _pick_chunk(epw: int) -> int:
    for k in (16, 40, 32, 24, 8):
        if epw % k == 0:
            return k
    return 0


def _sc_scatter_body(x_hbm, ei_hbm, acc0_hbm, acc1_hbm,
                     idx_s, idx_d, rows, acc_sh, gsem, ssem, *,
                     n_rows, e_total, epw, k_chunk, nbuf):
    c = lax.axis_index("c")
    s = lax.axis_index("s")
    wid = c * NS + s
    rpt = (n_rows // NS) // 8 * 8
    tail = n_rows - NS * rpt
    r0 = s * rpt
    t0 = NS * rpt

    def _part_copy(get_src, get_dst):
        pltpu.sync_copy(get_src(pl.ds(r0, rpt)), get_dst(pl.ds(r0, rpt)))
        if tail:
            @pl.when(s == NS - 1)
            def _():
                pltpu.sync_copy(get_src(pl.ds(t0, tail)),
                                get_dst(pl.ds(t0, tail)))

    base = wid * epw
    n_chunks = epw // k_chunk
    n_groups = n_chunks // nbuf
    n_tail = n_chunks % nbuf
    pltpu.sync_copy(ei_hbm.at[pl.ds(base, epw)], idx_s)
    pltpu.sync_copy(ei_hbm.at[pl.ds(e_total + base, epw)], idx_d)

    def gdesc(i, b):
        return pltpu.make_async_copy(
            x_hbm.at[idx_s.at[pl.ds(i * k_chunk, k_chunk)]],
            rows.at[b], gsem.at[b])

    def sdesc(i, b):
        return pltpu.make_async_copy(
            rows.at[b],
            acc_sh.at[idx_d.at[pl.ds(i * k_chunk, k_chunk)]],
            ssem.at[b])

    @pl.when(c == 0)
    def _():
        for b in range(nbuf):
            gdesc(b, b).start()
        _part_copy(lambda d: x_hbm.at[d], lambda d: acc_sh.at[d])

    @pl.when(c != 0)
    def _():
        for b in range(1, nbuf):
            gdesc(b, b).start()
        zvec = jnp.zeros((16,), jnp.float32)

        def zrow(r, _):
            for j in range(0, 128, 16):
                rows[0, r, pl.ds(j, 16)] = zvec
            return 0

        lax.fori_loop(0, k_chunk, zrow, 0)
        n_slabs = n_rows // k_chunk
        for t in range(-(-n_slabs // NS)):
            slab = s + NS * t

            @pl.when(slab < n_slabs)
            def _():
                pltpu.sync_copy(
                    rows.at[0],
                    acc_sh.at[pl.ds(slab * k_chunk, k_chunk)])

        gdesc(0, 0).start()

    plsc.subcore_barrier()

    def body(g, _):
        for b in range(nbuf):
            i = g * nbuf + b
            gdesc(i, b).wait()
            sdesc(i, b).start(add=True)
        for b in range(nbuf):
            i = (g + 1) * nbuf + b

            @pl.when(i < n_chunks)
            def _():
                sdesc(i - nbuf, b).wait()
                gdesc(i, b).start()
        return 0

    lax.fori_loop(0, n_groups, body, 0)
    for t in range(n_tail):
        i = n_groups * nbuf + t
        gdesc(i, t).wait()
        sdesc(i, t).start(add=True)
    for i in range(n_chunks - nbuf, n_chunks):
        sdesc(i, i % nbuf).wait()

    plsc.subcore_barrier()

    @pl.when(c == 0)
    def _():
        _part_copy(lambda d: acc_sh.at[d], lambda d: acc0_hbm.at[d])

    @pl.when(c != 0)
    def _():
        _part_copy(lambda d: acc_sh.at[d], lambda d: acc1_hbm.at[d])


def _tc_body(acc0_ref, acc1_ref, w_ref, p_ref, out_ref):
    hp = acc0_ref[:] + acc1_ref[:]
    h = lax.dot_general(hp, w_ref[:], (((1,), (1,)), ((), ())),
                        preferred_element_type=jnp.float32)
    b = p_ref[0:1, :]
    gamma = p_ref[1:2, :]
    beta = p_ref[2:3, :]
    mean = p_ref[3:4, :]
    var = p_ref[4:5, :]
    h = jnp.maximum(h + b, 0.0)
    scale = gamma * lax.rsqrt(var + BN_EPS)
    out_ref[:] = h * scale + (beta - mean * scale)


def kernel(x, edge_index, W, b, gamma, beta, running_mean, running_var):
    N, D = x.shape
    E = edge_index.shape[1]
    ei_flat = edge_index.reshape(-1)

    n_workers = NC * NS
    assert E % n_workers == 0, E
    epw = E // n_workers
    k_chunk = _pick_chunk(epw)
    assert k_chunk > 0 and N % k_chunk == 0, (epw, N)

    spmem_words = 2097151 - 4000 - N * D
    per_tile = spmem_words // NS
    nbuf = min(12, (per_tile - 2 * epw) // (k_chunk * D))
    assert nbuf >= 2, (per_tile, epw, k_chunk)

    mesh = plsc.VectorSubcoreMesh(core_axis_name="c", subcore_axis_name="s")

    sc = pl.kernel(
        functools.partial(_sc_scatter_body, n_rows=N, e_total=E, epw=epw,
                          k_chunk=k_chunk, nbuf=nbuf),
        out_type=(jax.ShapeDtypeStruct((N, D), jnp.float32),
                  jax.ShapeDtypeStruct((N, D), jnp.float32)),
        mesh=mesh,
        scratch_types=[
            pltpu.VMEM((epw,), jnp.int32),
            pltpu.VMEM((epw,), jnp.int32),
            pltpu.VMEM((nbuf, k_chunk, D), jnp.float32),
            pltpu.VMEM_SHARED((N, D), jnp.float32),
            pltpu.SemaphoreType.DMA((nbuf,)),
            pltpu.SemaphoreType.DMA((nbuf,)),
        ],
    )
    acc0, acc1 = sc(x, ei_flat)

    params = jnp.stack([b, gamma, beta, running_mean, running_var] +
                       [jnp.zeros_like(b)] * 3)

    blk = 1000
    assert N % blk == 0
    h = pl.pallas_call(
        _tc_body,
        grid=(N // blk,),
        in_specs=[
            pl.BlockSpec((blk, D), lambda i: (i, 0)),
            pl.BlockSpec((blk, D), lambda i: (i, 0)),
            pl.BlockSpec((D, D), lambda i: (0, 0)),
            pl.BlockSpec((8, D), lambda i: (0, 0)),
        ],
        out_specs=pl.BlockSpec((blk, D), lambda i: (i, 0)),
        out_shape=jax.ShapeDtypeStruct((N, D), jnp.float32),
    )(acc0, acc1, W, params)

    return (h, h)


# speedup vs baseline: 1.1244x; 1.1244x over previous
import functools

import jax
import jax.numpy as jnp
from jax import lax
from jax.experimental import pallas as pl
from jax.experimental.pallas import tpu as pltpu
from jax.experimental.pallas import tpu_sc as plsc

BN_EPS = 1e-5
NC = 2
NS = 16


def _pick_chunk(epw: int) -> int:
    for k in (40, 32, 24, 16, 8):
        if epw % k == 0:
            return k
    return 0


def _sc_scatter_body(x_hbm, ei_hbm, acc0_hbm, acc1_hbm,
                     idx_s, idx_d, rows, acc_sh, gsem, ssem, *,
                     n_rows, e_total, epw, k_chunk, nbuf):
    c = lax.axis_index("c")
    s = lax.axis_index("s")
    wid = c * NS + s
    rpt = (n_rows // NS) // 8 * 8
    tail = n_rows - NS * rpt
    r0 = s * rpt
    t0 = NS * rpt

    def _part_copy(get_src, get_dst):
        pltpu.sync_copy(get_src(pl.ds(r0, rpt)), get_dst(pl.ds(r0, rpt)))
        if tail:
            @pl.when(s == NS - 1)
            def _():
                pltpu.sync_copy(get_src(pl.ds(t0, tail)),
                                get_dst(pl.ds(t0, tail)))

    base = wid * epw
    n_chunks = epw // k_chunk
    n_groups = n_chunks // nbuf
    n_tail = n_chunks % nbuf
    pltpu.sync_copy(ei_hbm.at[pl.ds(base, epw)], idx_s)
    pltpu.sync_copy(ei_hbm.at[pl.ds(e_total + base, epw)], idx_d)

    def gdesc(i, b):
        return pltpu.make_async_copy(
            x_hbm.at[idx_s.at[pl.ds(i * k_chunk, k_chunk)]],
            rows.at[b], gsem.at[b])

    def sdesc(i, b):
        return pltpu.make_async_copy(
            rows.at[b],
            acc_sh.at[idx_d.at[pl.ds(i * k_chunk, k_chunk)]],
            ssem.at[b])

    @pl.when(c == 0)
    def _():
        for b in range(nbuf):
            gdesc(b, b).start()
        _part_copy(lambda d: x_hbm.at[d], lambda d: acc_sh.at[d])

    @pl.when(c != 0)
    def _():
        for b in range(1, nbuf):
            gdesc(b, b).start()
        zvec = jnp.zeros((16,), jnp.float32)

        def zrow(r, _):
            for j in range(0, 128, 16):
                rows[0, r, pl.ds(j, 16)] = zvec
            return 0

        lax.fori_loop(0, k_chunk, zrow, 0)
        n_slabs = n_rows // k_chunk
        for t in range(-(-n_slabs // NS)):
            slab = s + NS * t

            @pl.when(slab < n_slabs)
            def _():
                pltpu.sync_copy(
                    rows.at[0],
                    acc_sh.at[pl.ds(slab * k_chunk, k_chunk)])

        gdesc(0, 0).start()

    plsc.subcore_barrier()

    def body(g, _):
        for b in range(nbuf):
            i = g * nbuf + b
            gdesc(i, b).wait()
            sdesc(i, b).start(add=True)
        for b in range(nbuf):
            i = (g + 1) * nbuf + b

            @pl.when(i < n_chunks)
            def _():
                sdesc(i - nbuf, b).wait()
                gdesc(i, b).start()
        return 0

    lax.fori_loop(0, n_groups, body, 0)
    for t in range(n_tail):
        i = n_groups * nbuf + t
        gdesc(i, t).wait()
        sdesc(i, t).start(add=True)
    for i in range(n_chunks - nbuf, n_chunks):
        sdesc(i, i % nbuf).wait()

    plsc.subcore_barrier()

    @pl.when(c == 0)
    def _():
        _part_copy(lambda d: acc_sh.at[d], lambda d: acc0_hbm.at[d])

    @pl.when(c != 0)
    def _():
        _part_copy(lambda d: acc_sh.at[d], lambda d: acc1_hbm.at[d])


def _tc_body(acc0_ref, acc1_ref, w_ref, p_ref, out_ref):
    hp = acc0_ref[:] + acc1_ref[:]
    h = lax.dot_general(hp, w_ref[:], (((1,), (1,)), ((), ())),
                        preferred_element_type=jnp.float32)
    b = p_ref[0:1, :]
    gamma = p_ref[1:2, :]
    beta = p_ref[2:3, :]
    mean = p_ref[3:4, :]
    var = p_ref[4:5, :]
    h = jnp.maximum(h + b, 0.0)
    scale = gamma * lax.rsqrt(var + BN_EPS)
    out_ref[:] = h * scale + (beta - mean * scale)


def kernel(x, edge_index, W, b, gamma, beta, running_mean, running_var):
    N, D = x.shape
    E = edge_index.shape[1]
    ei_flat = edge_index.reshape(-1)

    n_workers = NC * NS
    assert E % n_workers == 0, E
    epw = E // n_workers
    k_chunk = _pick_chunk(epw)
    assert k_chunk > 0 and N % k_chunk == 0, (epw, N)

    spmem_words = 2097151 - 4000 - N * D
    per_tile = spmem_words // NS
    nbuf = min(6, (per_tile - 2 * epw) // (k_chunk * D))
    assert nbuf >= 2, (per_tile, epw, k_chunk)

    mesh = plsc.VectorSubcoreMesh(core_axis_name="c", subcore_axis_name="s")

    sc = pl.kernel(
        functools.partial(_sc_scatter_body, n_rows=N, e_total=E, epw=epw,
                          k_chunk=k_chunk, nbuf=nbuf),
        out_type=(jax.ShapeDtypeStruct((N, D), jnp.float32),
                  jax.ShapeDtypeStruct((N, D), jnp.float32)),
        mesh=mesh,
        scratch_types=[
            pltpu.VMEM((epw,), jnp.int32),
            pltpu.VMEM((epw,), jnp.int32),
            pltpu.VMEM((nbuf, k_chunk, D), jnp.float32),
            pltpu.VMEM_SHARED((N, D), jnp.float32),
            pltpu.SemaphoreType.DMA((nbuf,)),
            pltpu.SemaphoreType.DMA((nbuf,)),
        ],
    )
    acc0, acc1 = sc(x, ei_flat)

    params = jnp.stack([b, gamma, beta, running_mean, running_var] +
                       [jnp.zeros_like(b)] * 3)

    blk = 2000
    assert N % blk == 0
    h = pl.pallas_call(
        _tc_body,
        grid=(N // blk,),
        in_specs=[
            pl.BlockSpec((blk, D), lambda i: (i, 0)),
            pl.BlockSpec((blk, D), lambda i: (i, 0)),
            pl.BlockSpec((D, D), lambda i: (0, 0)),
            pl.BlockSpec((8, D), lambda i: (0, 0)),
        ],
        out_specs=pl.BlockSpec((blk, D), lambda i: (i, 0)),
        out_shape=jax.ShapeDtypeStruct((N, D), jnp.float32),
    )(acc0, acc1, W, params)

    return (h, h)


# speedup vs baseline: 1.1429x; 1.0165x over previous
import functools

import jax
import jax.numpy as jnp
from jax import lax
from jax.experimental import pallas as pl
from jax.experimental.pallas import tpu as pltpu
from jax.experimental.pallas import tpu_sc as plsc

BN_EPS = 1e-5
NC = 2
NS = 16


def _pick_chunk(epw: int) -> int:
    for k in (40, 32, 24, 16, 8):
        if epw % k == 0:
            return k
    return 0


def _sc_scatter_body(x_hbm, ei_hbm, acc0_hbm, acc1_hbm,
                     idx_s, idx_d, rows, acc_sh, gsem, ssem, *,
                     n_rows, e_total, epw, k_chunk, nbuf):
    c = lax.axis_index("c")
    s = lax.axis_index("s")
    wid = c * NS + s
    rpt = (n_rows // NS) // 8 * 8
    tail = n_rows - NS * rpt
    r0 = s * rpt
    t0 = NS * rpt

    def _part_copy(get_src, get_dst):
        pltpu.sync_copy(get_src(pl.ds(r0, rpt)), get_dst(pl.ds(r0, rpt)))
        if tail:
            @pl.when(s == NS - 1)
            def _():
                pltpu.sync_copy(get_src(pl.ds(t0, tail)),
                                get_dst(pl.ds(t0, tail)))

    base = wid * epw
    n_chunks = epw // k_chunk
    n_groups = n_chunks // nbuf
    n_tail = n_chunks % nbuf
    pltpu.sync_copy(ei_hbm.at[pl.ds(base, epw)], idx_s)
    pltpu.sync_copy(ei_hbm.at[pl.ds(e_total + base, epw)], idx_d)

    def gdesc(i, b):
        return pltpu.make_async_copy(
            x_hbm.at[idx_s.at[pl.ds(i * k_chunk, k_chunk)]],
            rows.at[b], gsem.at[b])

    def sdesc(i, b):
        return pltpu.make_async_copy(
            rows.at[b],
            acc_sh.at[idx_d.at[pl.ds(i * k_chunk, k_chunk)]],
            ssem.at[b])

    @pl.when(c == 0)
    def _():
        for b in range(nbuf):
            gdesc(b, b).start()
        _part_copy(lambda d: x_hbm.at[d], lambda d: acc_sh.at[d])

    @pl.when(c != 0)
    def _():
        for b in range(1, nbuf):
            gdesc(b, b).start()
        zvec = jnp.zeros((16,), jnp.float32)

        def zrow(r, _):
            for j in range(0, 128, 16):
                rows[0, r, pl.ds(j, 16)] = zvec
            return 0

        lax.fori_loop(0, k_chunk, zrow, 0)
        n_slabs = n_rows // k_chunk
        for t in range(-(-n_slabs // NS)):
            slab = s + NS * t

            @pl.when(slab < n_slabs)
            def _():
                pltpu.sync_copy(
                    rows.at[0],
                    acc_sh.at[pl.ds(slab * k_chunk, k_chunk)])

        gdesc(0, 0).start()

    plsc.subcore_barrier()

    def body(g, _):
        for b in range(nbuf):
            i = g * nbuf + b
            gdesc(i, b).wait()
            sdesc(i, b).start(add=True)
        for b in range(nbuf):
            i = (g + 1) * nbuf + b

            @pl.when(i < n_chunks)
            def _():
                sdesc(i - nbuf, b).wait()
                gdesc(i, b).start()
        return 0

    lax.fori_loop(0, n_groups, body, 0)
    for t in range(n_tail):
        i = n_groups * nbuf + t
        gdesc(i, t).wait()
        sdesc(i, t).start(add=True)
    for i in range(n_chunks - nbuf, n_chunks):
        sdesc(i, i % nbuf).wait()

    plsc.subcore_barrier()

    @pl.when(c == 0)
    def _():
        _part_copy(lambda d: acc_sh.at[d], lambda d: acc0_hbm.at[d])

    @pl.when(c != 0)
    def _():
        _part_copy(lambda d: acc_sh.at[d], lambda d: acc1_hbm.at[d])


def _tc_body(acc0_ref, acc1_ref, w_ref, p_ref, out_ref):
    hp = acc0_ref[:] + acc1_ref[:]
    h = lax.dot_general(hp, w_ref[:], (((1,), (1,)), ((), ())),
                        preferred_element_type=jnp.float32)
    b = p_ref[0:1, :]
    gamma = p_ref[1:2, :]
    beta = p_ref[2:3, :]
    mean = p_ref[3:4, :]
    var = p_ref[4:5, :]
    h = jnp.maximum(h + b, 0.0)
    scale = gamma * lax.rsqrt(var + BN_EPS)
    out_ref[:] = h * scale + (beta - mean * scale)


def kernel(x, edge_index, W, b, gamma, beta, running_mean, running_var):
    N, D = x.shape
    E = edge_index.shape[1]
    ei_flat = edge_index.reshape(-1)

    n_workers = NC * NS
    assert E % n_workers == 0, E
    epw = E // n_workers
    k_chunk = _pick_chunk(epw)
    assert k_chunk > 0 and N % k_chunk == 0, (epw, N)

    spmem_words = 2097151 - 4000 - N * D
    per_tile = spmem_words // NS
    nbuf = min(6, (per_tile - 2 * epw) // (k_chunk * D))
    assert nbuf >= 2, (per_tile, epw, k_chunk)

    mesh = plsc.VectorSubcoreMesh(core_axis_name="c", subcore_axis_name="s")

    sc = pl.kernel(
        functools.partial(_sc_scatter_body, n_rows=N, e_total=E, epw=epw,
                          k_chunk=k_chunk, nbuf=nbuf),
        out_type=(jax.ShapeDtypeStruct((N, D), jnp.float32),
                  jax.ShapeDtypeStruct((N, D), jnp.float32)),
        mesh=mesh,
        scratch_types=[
            pltpu.VMEM((epw,), jnp.int32),
            pltpu.VMEM((epw,), jnp.int32),
            pltpu.VMEM((nbuf, k_chunk, D), jnp.float32),
            pltpu.VMEM_SHARED((N, D), jnp.float32),
            pltpu.SemaphoreType.DMA((nbuf,)),
            pltpu.SemaphoreType.DMA((nbuf,)),
        ],
    )
    acc0, acc1 = sc(x, ei_flat)

    params = jnp.stack([b, gamma, beta, running_mean, running_var] +
                       [jnp.zeros_like(b)] * 3)

    blk = 5000
    assert N % blk == 0
    h = pl.pallas_call(
        _tc_body,
        grid=(N // blk,),
        in_specs=[
            pl.BlockSpec((blk, D), lambda i: (i, 0)),
            pl.BlockSpec((blk, D), lambda i: (i, 0)),
            pl.BlockSpec((D, D), lambda i: (0, 0)),
            pl.BlockSpec((8, D), lambda i: (0, 0)),
        ],
        out_specs=pl.BlockSpec((blk, D), lambda i: (i, 0)),
        out_shape=jax.ShapeDtypeStruct((N, D), jnp.float32),
    )(acc0, acc1, W, params)

    return (h, h)
